# baseline (speedup 1.0000x reference)
import functools

import jax
import jax.numpy as jnp
from jax import lax
from jax.experimental import pallas as pl
from jax.experimental.pallas import tpu as pltpu
from jax.experimental.pallas import tpu_sc as plsc

_N = 10000
_E = 320000
_D = 128
_NCLASS = 16

_QR = 2528
_NB = 4
_ACC_ROWS = _QR + 8

_EPT = _E // 32
_K = 128
_CAP = 10240
_NCHMAX = _CAP // _K
_LISTLEN = _NB * 32 * _CAP
_CNTLEN = _NB * 32 * 16

_BN = 1000


def _sc_partition_build():
    mesh = plsc.VectorSubcoreMesh(core_axis_name="c", subcore_axis_name="s")

    @functools.partial(
        pl.kernel,
        out_type=[
            jax.ShapeDtypeStruct((_LISTLEN,), jnp.int32),
            jax.ShapeDtypeStruct((_LISTLEN,), jnp.int32),
            jax.ShapeDtypeStruct((_CNTLEN,), jnp.int32),
        ],
        mesh=mesh,
        scratch_types=[
            pltpu.VMEM((_EPT,), jnp.int32),
            pltpu.VMEM((_EPT,), jnp.int32),
            [pltpu.VMEM((_CAP + 16,), jnp.int32)] * _NB,
            [pltpu.VMEM((_CAP + 16,), jnp.int32)] * _NB,
            pltpu.VMEM((16,), jnp.int32),
        ],
        compiler_params=pltpu.CompilerParams(use_tc_tiling_on_sc=False, needs_layout_passes=False),
    )
    def sc_partition(src_hbm, dst_hbm, osrc_hbm, odst_hbm, ocnt_hbm,
                     src_v, dst_v, csrc_v, cdst_v, cnt_v):
        c = lax.axis_index("c")
        s = lax.axis_index("s")
        wid = s * 2 + c

        pltpu.sync_copy(src_hbm.at[pl.ds(wid * _EPT, _EPT)], src_v)
        pltpu.sync_copy(dst_hbm.at[pl.ds(wid * _EPT, _EPT)], dst_v)

        def body(i, offs):
            sv = src_v[pl.ds(i * 16, 16)]
            dv = dst_v[pl.ds(i * 16, 16)]
            new_offs = []
            for b in range(_NB):
                lo = b * _QR
                if b == 0:
                    msk = dv < _QR
                elif b == _NB - 1:
                    msk = dv >= lo
                else:
                    msk = jnp.logical_and(dv >= lo, dv < lo + _QR)
                one = jnp.full((16,), 1, jnp.int32)
                nul = jnp.zeros((16,), jnp.int32)
                cum = plsc.cumsum(jnp.where(msk, one, nul))
                pos = offs[b] + cum - 1
                plsc.store_scatter(csrc_v[b], [pos], sv, mask=msk)
                plsc.store_scatter(cdst_v[b], [pos], dv - lo, mask=msk)
                new_offs.append(offs[b] + cum[15])
            return tuple(new_offs)

        zero = jnp.int32(0)
        offs = lax.fori_loop(0, _EPT // 16, body, (zero, zero, zero, zero))

        lane = jnp.arange(16, dtype=jnp.int32)
        pad_src = jnp.zeros((16,), jnp.int32)
        pad_dst = _QR + (lane & 7)
        for b in range(_NB):
            off = offs[b]
            nch = (off + _K - 1) // _K
            n4 = ((nch + 3) // 4) * 4
            lim = n4 * _K
            for kk in range(32):
                st = off + 16 * kk

                @pl.when(st < lim)
                def _():
                    csrc_v[b][pl.ds(st, 16)] = pad_src
                    cdst_v[b][pl.ds(st, 16)] = pad_dst

            base = (b * 32 + wid) * _CAP
            pltpu.sync_copy(csrc_v[b].at[pl.ds(0, _CAP)],
                            osrc_hbm.at[pl.ds(base, _CAP)])
            pltpu.sync_copy(cdst_v[b].at[pl.ds(0, _CAP)],
                            odst_hbm.at[pl.ds(base, _CAP)])
            cnt_v[...] = jnp.where(lane == 0, n4, 0)
            pltpu.sync_copy(
                cnt_v, ocnt_hbm.at[pl.ds((b * 32 + wid) * 16, 16)])

    return sc_partition


def _sc_scatter_build():
    mesh = plsc.VectorSubcoreMesh(core_axis_name="c", subcore_axis_name="s")

    @functools.partial(
        pl.kernel,
        out_type=jax.ShapeDtypeStruct((_NB, _ACC_ROWS, _D), jnp.float32),
        mesh=mesh,
        scratch_types=[
            pltpu.VMEM((2, _NCHMAX, _K), jnp.int32),
            pltpu.VMEM((2, _NCHMAX, _K), jnp.int32),
            pltpu.VMEM((2, _K, _D), jnp.float32),
            pltpu.VMEM((80, _D), jnp.float32),
            pltpu.VMEM((16,), jnp.int32),
            pltpu.VMEM_SHARED((_ACC_ROWS, _D), jnp.float32),
            [pltpu.SemaphoreType.DMA] * 2,
        ],
        compiler_params=pltpu.CompilerParams(use_tc_tiling_on_sc=False, needs_layout_passes=False),
    )
    def sc_scatter(m_hbm, srcl_hbm, dstl_hbm, cnt_hbm, out_hbm,
                   src_v, dst_v, rows_v, z_v, cnt_v, acc_sh, gsems):
        c = lax.axis_index("c")
        s = lax.axis_index("s")

        zero = jnp.zeros((16,), jnp.float32)

        def zrow(r, carry):
            for cc in range(_D // 16):
                z_v[r, pl.ds(cc * 16, 16)] = zero
            return carry

        lax.fori_loop(0, 80, zrow, 0)

        for k in range(2):
            b = 2 * k + c

            @pl.when(s < 15)
            def _():
                pltpu.sync_copy(z_v, acc_sh.at[pl.ds(s * 160, 80)])
                pltpu.sync_copy(z_v, acc_sh.at[pl.ds(s * 160 + 80, 80)])

            @pl.when(s == 15)
            def _():
                pltpu.sync_copy(z_v, acc_sh.at[pl.ds(15 * 160, 80)])
                pltpu.sync_copy(z_v.at[pl.ds(0, _ACC_ROWS - 15 * 160 - 80)],
                                acc_sh.at[pl.ds(15 * 160 + 80,
                                                _ACC_ROWS - 15 * 160 - 80)])

            plsc.subcore_barrier()

            for t in range(2):
                p = 2 * s + t
                pltpu.sync_copy(
                    srcl_hbm.at[b, p], src_v.at[t])
                pltpu.sync_copy(
                    dstl_hbm.at[b, p], dst_v.at[t])
                pltpu.sync_copy(
                    cnt_hbm.at[pl.ds((b * 32 + p) * 16, 16)], cnt_v)
                n = cnt_v[...][0]

                def body(jj, carry):
                    j0 = jj * 2
                    for u in range(2):
                        pltpu.async_copy(
                            m_hbm.at[src_v.at[t, j0 + u]], rows_v.at[u],
                            gsems[u])
                    for u in range(2):
                        pltpu.make_async_copy(
                            m_hbm.at[src_v.at[t, j0 + u]], rows_v.at[u],
                            gsems[u]).wait()
                        pltpu.sync_copy(
                            rows_v.at[u], acc_sh.at[dst_v.at[t, j0 + u]],
                            add=True)
                    return carry

                lax.fori_loop(0, n // 2, body, 0)

            plsc.subcore_barrier()

            @pl.when(s < 15)
            def _():
                pltpu.sync_copy(acc_sh.at[pl.ds(s * 160, 160)],
                                out_hbm.at[b, pl.ds(s * 160, 160)])

            @pl.when(s == 15)
            def _():
                pltpu.sync_copy(
                    acc_sh.at[pl.ds(15 * 160, _ACC_ROWS - 15 * 160)],
                    out_hbm.at[b, pl.ds(15 * 160, _ACC_ROWS - 15 * 160)])

            plsc.subcore_barrier()

    return sc_scatter


_sc_partition = _sc_partition_build()
_sc_scatter = _sc_scatter_build()


def _edge_mm_body(h_ref, W_ref, b_ref, m_ref):
    m_ref[...] = (
        jnp.dot(h_ref[...], W_ref[...], preferred_element_type=jnp.float32)
        + b_ref[...]
    )


def _gru(a_ref, h_ref, W_ih_ref, W_hh_ref, b_ih_ref, b_hh_ref):
    a = a_ref[...]
    gi = jnp.dot(a, W_ih_ref[...], preferred_element_type=jnp.float32) + b_ih_ref[...]
    h = h_ref[...]
    gh = jnp.dot(h, W_hh_ref[...], preferred_element_type=jnp.float32) + b_hh_ref[...]
    r = jax.nn.sigmoid(gi[:, :_D] + gh[:, :_D])
    z = jax.nn.sigmoid(gi[:, _D:2 * _D] + gh[:, _D:2 * _D])
    n = jnp.tanh(gi[:, 2 * _D:] + r * gh[:, 2 * _D:])
    return (1.0 - z) * n + z * h


def _gru_edge_body(a_ref, h_ref, W_ih_ref, W_hh_ref, b_ih_ref, b_hh_ref,
                   W_edge_ref, b_edge_ref, hn_ref, m_ref):
    hn = _gru(a_ref, h_ref, W_ih_ref, W_hh_ref, b_ih_ref, b_hh_ref)
    hn_ref[...] = hn
    m_ref[...] = (
        jnp.dot(hn, W_edge_ref[...], preferred_element_type=jnp.float32)
        + b_edge_ref[...]
    )


def _fc_body(h_ref, W_fc_ref, b_fc_ref, out_ref):
    hn = h_ref[...]
    e = jnp.where(hn > 0, hn, jnp.exp(jnp.minimum(hn, 0.0)) - 1.0)
    out_ref[...] = (
        jnp.dot(e, W_fc_ref[...], preferred_element_type=jnp.float32)
        + b_fc_ref[...]
    )


def _full(shape):
    return pl.BlockSpec(shape, lambda i: tuple(0 for _ in shape))


_GRID = _N // _BN

_edge_mm = pl.pallas_call(
    _edge_mm_body,
    grid=(_GRID,),
    in_specs=[
        pl.BlockSpec((_BN, _D), lambda i: (i, 0)),
        _full((_D, _D)),
        _full((1, _D)),
    ],
    out_specs=pl.BlockSpec((_BN, _D), lambda i: (i, 0)),
    out_shape=jax.ShapeDtypeStruct((_N, _D), jnp.float32),
)

_gru_edge = pl.pallas_call(
    _gru_edge_body,
    grid=(_GRID,),
    in_specs=[
        pl.BlockSpec((_BN, _D), lambda i: (i, 0)),
        pl.BlockSpec((_BN, _D), lambda i: (i, 0)),
        _full((_D, 3 * _D)),
        _full((_D, 3 * _D)),
        _full((1, 3 * _D)),
        _full((1, 3 * _D)),
        _full((_D, _D)),
        _full((1, _D)),
    ],
    out_specs=[
        pl.BlockSpec((_BN, _D), lambda i: (i, 0)),
        pl.BlockSpec((_BN, _D), lambda i: (i, 0)),
    ],
    out_shape=[
        jax.ShapeDtypeStruct((_N, _D), jnp.float32),
        jax.ShapeDtypeStruct((_N, _D), jnp.float32),
    ],
)

_fc_head = pl.pallas_call(
    _fc_body,
    grid=(_GRID,),
    in_specs=[
        pl.BlockSpec((_BN, _D), lambda i: (i, 0)),
        _full((_D, _NCLASS)),
        _full((1, _NCLASS)),
    ],
    out_specs=pl.BlockSpec((_BN, _NCLASS), lambda i: (i, 0)),
    out_shape=jax.ShapeDtypeStruct((_N, _NCLASS), jnp.float32),
)


def kernel(x, edge_index, W_edge, b_edge, W_ih, W_hh, b_ih, b_hh, W_fc, b_fc):
    src = edge_index[0].astype(jnp.int32)
    dst = edge_index[1].astype(jnp.int32)
    b_edge2 = b_edge.reshape(1, _D)
    b_ih2 = b_ih.reshape(1, 3 * _D)
    b_hh2 = b_hh.reshape(1, 3 * _D)
    b_fc2 = b_fc.reshape(1, _NCLASS)

    srcl, dstl, cnts = _sc_partition(src, dst)
    srcl = srcl.reshape(_NB, 32, _NCHMAX, _K)
    dstl = dstl.reshape(_NB, 32, _NCHMAX, _K)

    m1 = _edge_mm(x, W_edge, b_edge2)

    def layer(carry, _):
        h, m = carry
        p = _sc_scatter(m, srcl, dstl, cnts)
        a = p[:, :_QR, :].reshape(_NB * _QR, _D)[:_N]
        hn, mn = _gru_edge(a, h, W_ih, W_hh, b_ih2, b_hh2, W_edge, b_edge2)
        return (hn, mn), None

    (h2, _), _ = lax.scan(layer, (x, m1), None, length=2)
    logits = _fc_head(h2, W_fc, b_fc2)
    return logits


# speedup vs baseline: 2.6794x; 2.6794x over previous
import functools

import jax
import jax.numpy as jnp
from jax import lax
from jax.experimental import pallas as pl
from jax.experimental.pallas import tpu as pltpu
from jax.experimental.pallas import tpu_sc as plsc

_N = 10000
_E = 320000
_D = 128
_DH = 64
_NCLASS = 16

_NCH = 200
_K = 100
_NPAD = 10112
_ROWS_PER_TILE = _NPAD // 16

_BN = 1000


def _sc_scatter_build():
    mesh = plsc.VectorSubcoreMesh(core_axis_name="c", subcore_axis_name="s")

    @functools.partial(
        pl.kernel,
        out_type=jax.ShapeDtypeStruct((2, _NPAD, _DH), jnp.float32),
        mesh=mesh,
        scratch_types=[
            pltpu.VMEM((_NCH, _K), jnp.int32),
            pltpu.VMEM((_NCH, _K), jnp.int32),
            pltpu.VMEM((_K, _DH), jnp.float32),
            pltpu.VMEM((_ROWS_PER_TILE, _DH), jnp.float32),
            pltpu.VMEM_SHARED((_NPAD, _DH), jnp.float32),
            pltpu.SemaphoreType.DMA,
        ],
        compiler_params=pltpu.CompilerParams(use_tc_tiling_on_sc=False),
    )
    def sc_scatter(m2_hbm, src_hbm, dst_hbm, out_hbm,
                   src_v, dst_v, rows_v, z_v, acc_sh, gsem):
        c = lax.axis_index("c")
        s = lax.axis_index("s")

        pltpu.sync_copy(src_hbm.at[s], src_v)
        pltpu.sync_copy(dst_hbm.at[s], dst_v)

        zero = jnp.zeros((16,), jnp.float32)

        def zrow(r, carry):
            for cc in range(_DH // 16):
                z_v[r, pl.ds(cc * 16, 16)] = zero
            return carry

        lax.fori_loop(0, _ROWS_PER_TILE, zrow, 0)

        row0 = s * _ROWS_PER_TILE
        pltpu.sync_copy(z_v, acc_sh.at[pl.ds(row0, _ROWS_PER_TILE)])
        plsc.subcore_barrier()

        m_ref = m2_hbm.at[c]

        def body(j, carry):
            pltpu.async_copy(m_ref.at[src_v.at[j]], rows_v, gsem).wait()
            pltpu.sync_copy(rows_v, acc_sh.at[dst_v.at[j]], add=True)
            return carry

        lax.fori_loop(0, _NCH, body, 0)

        plsc.subcore_barrier()

        pltpu.sync_copy(acc_sh.at[pl.ds(row0, _ROWS_PER_TILE)],
                        out_hbm.at[c, pl.ds(row0, _ROWS_PER_TILE)])

    return sc_scatter


_sc_scatter = _sc_scatter_build()


def _edge_mm_body(h_ref, W_ref, b_ref, m2_ref):
    m = (
        jnp.dot(h_ref[...], W_ref[...], preferred_element_type=jnp.float32)
        + b_ref[...]
    )
    m2_ref[0] = m[:, :_DH]
    m2_ref[1] = m[:, _DH:]


def _gru(p_ref, h_ref, W_ih_ref, W_hh_ref, b_ih_ref, b_hh_ref):
    a_lo = p_ref[0]
    a_hi = p_ref[1]
    gi = (
        jnp.dot(a_lo, W_ih_ref[...][:_DH, :], preferred_element_type=jnp.float32)
        + jnp.dot(a_hi, W_ih_ref[...][_DH:, :], preferred_element_type=jnp.float32)
        + b_ih_ref[...]
    )
    h = h_ref[...]
    gh = jnp.dot(h, W_hh_ref[...], preferred_element_type=jnp.float32) + b_hh_ref[...]
    r = jax.nn.sigmoid(gi[:, :_D] + gh[:, :_D])
    z = jax.nn.sigmoid(gi[:, _D:2 * _D] + gh[:, _D:2 * _D])
    n = jnp.tanh(gi[:, 2 * _D:] + r * gh[:, 2 * _D:])
    return (1.0 - z) * n + z * h


def _gru_edge_body(p_ref, h_ref, W_ih_ref, W_hh_ref, b_ih_ref, b_hh_ref,
                   W_edge_ref, b_edge_ref, hn_ref, m2_ref):
    hn = _gru(p_ref, h_ref, W_ih_ref, W_hh_ref, b_ih_ref, b_hh_ref)
    hn_ref[...] = hn
    m = (
        jnp.dot(hn, W_edge_ref[...], preferred_element_type=jnp.float32)
        + b_edge_ref[...]
    )
    m2_ref[0] = m[:, :_DH]
    m2_ref[1] = m[:, _DH:]


def _fc_body(h_ref, W_fc_ref, b_fc_ref, out_ref):
    hn = h_ref[...]
    e = jnp.where(hn > 0, hn, jnp.exp(jnp.minimum(hn, 0.0)) - 1.0)
    out_ref[...] = (
        jnp.dot(e, W_fc_ref[...], preferred_element_type=jnp.float32)
        + b_fc_ref[...]
    )


def _full(shape):
    return pl.BlockSpec(shape, lambda i: tuple(0 for _ in shape))


_GRID = _N // _BN

_edge_mm = pl.pallas_call(
    _edge_mm_body,
    grid=(_GRID,),
    in_specs=[
        pl.BlockSpec((_BN, _D), lambda i: (i, 0)),
        _full((_D, _D)),
        _full((1, _D)),
    ],
    out_specs=pl.BlockSpec((2, _BN, _DH), lambda i: (0, i, 0)),
    out_shape=jax.ShapeDtypeStruct((2, _N, _DH), jnp.float32),
)

_gru_edge = pl.pallas_call(
    _gru_edge_body,
    grid=(_GRID,),
    in_specs=[
        pl.BlockSpec((2, _BN, _DH), lambda i: (0, i, 0)),
        pl.BlockSpec((_BN, _D), lambda i: (i, 0)),
        _full((_D, 3 * _D)),
        _full((_D, 3 * _D)),
        _full((1, 3 * _D)),
        _full((1, 3 * _D)),
        _full((_D, _D)),
        _full((1, _D)),
    ],
    out_specs=[
        pl.BlockSpec((_BN, _D), lambda i: (i, 0)),
        pl.BlockSpec((2, _BN, _DH), lambda i: (0, i, 0)),
    ],
    out_shape=[
        jax.ShapeDtypeStruct((_N, _D), jnp.float32),
        jax.ShapeDtypeStruct((2, _N, _DH), jnp.float32),
    ],
)

_fc_head = pl.pallas_call(
    _fc_body,
    grid=(_GRID,),
    in_specs=[
        pl.BlockSpec((_BN, _D), lambda i: (i, 0)),
        _full((_D, _NCLASS)),
        _full((1, _NCLASS)),
    ],
    out_specs=pl.BlockSpec((_BN, _NCLASS), lambda i: (i, 0)),
    out_shape=jax.ShapeDtypeStruct((_N, _NCLASS), jnp.float32),
)


def kernel(x, edge_index, W_edge, b_edge, W_ih, W_hh, b_ih, b_hh, W_fc, b_fc):
    src = edge_index[0].astype(jnp.int32).reshape(16, _NCH, _K)
    dst = edge_index[1].astype(jnp.int32).reshape(16, _NCH, _K)
    b_edge2 = b_edge.reshape(1, _D)
    b_ih2 = b_ih.reshape(1, 3 * _D)
    b_hh2 = b_hh.reshape(1, 3 * _D)
    b_fc2 = b_fc.reshape(1, _NCLASS)

    m1 = _edge_mm(x, W_edge, b_edge2)

    def layer(carry, _):
        h, m = carry
        p = _sc_scatter(m, src, dst)
        hn, mn = _gru_edge(p, h, W_ih, W_hh, b_ih2, b_hh2, W_edge, b_edge2)
        return (hn, mn), None

    (h2, _), _ = lax.scan(layer, (x, m1), None, length=2)
    logits = _fc_head(h2, W_fc, b_fc2)
    return logits


# speedup vs baseline: 3.0916x; 1.1538x over previous
import functools

import jax
import jax.numpy as jnp
from jax import lax
from jax.experimental import pallas as pl
from jax.experimental.pallas import tpu as pltpu
from jax.experimental.pallas import tpu_sc as plsc

_N = 10000
_E = 320000
_D = 128
_DH = 64
_NCLASS = 16

_NCH = 160
_K = 125
_NPAD = 10112
_ROWS_PER_TILE = _NPAD // 16

_BN = 1000


def _sc_scatter_build():
    mesh = plsc.VectorSubcoreMesh(core_axis_name="c", subcore_axis_name="s")

    @functools.partial(
        pl.kernel,
        out_type=jax.ShapeDtypeStruct((2, _NPAD, _DH), jnp.float32),
        mesh=mesh,
        scratch_types=[
            pltpu.VMEM((_NCH, _K), jnp.int32),
            pltpu.VMEM((_NCH, _K), jnp.int32),
            pltpu.VMEM((_K, _DH), jnp.float32),
            pltpu.VMEM((_ROWS_PER_TILE, _DH), jnp.float32),
            pltpu.VMEM_SHARED((_NPAD, _DH), jnp.float32),
            pltpu.SemaphoreType.DMA,
        ],
        compiler_params=pltpu.CompilerParams(use_tc_tiling_on_sc=False),
    )
    def sc_scatter(m2_hbm, src_hbm, dst_hbm, out_hbm,
                   src_v, dst_v, rows_v, z_v, acc_sh, gsem):
        c = lax.axis_index("c")
        s = lax.axis_index("s")

        pltpu.sync_copy(src_hbm.at[s], src_v)
        pltpu.sync_copy(dst_hbm.at[s], dst_v)

        zero = jnp.zeros((16,), jnp.float32)

        def zrow(r, carry):
            for cc in range(_DH // 16):
                z_v[r, pl.ds(cc * 16, 16)] = zero
            return carry

        lax.fori_loop(0, _ROWS_PER_TILE, zrow, 0)

        row0 = s * _ROWS_PER_TILE
        pltpu.sync_copy(z_v, acc_sh.at[pl.ds(row0, _ROWS_PER_TILE)])
        plsc.subcore_barrier()

        m_ref = m2_hbm.at[c]

        def body(j, carry):
            pltpu.async_copy(m_ref.at[src_v.at[j]], rows_v, gsem).wait()
            pltpu.sync_copy(rows_v, acc_sh.at[dst_v.at[j]], add=True)
            return carry

        lax.fori_loop(0, _NCH, body, 0)

        plsc.subcore_barrier()

        pltpu.sync_copy(acc_sh.at[pl.ds(row0, _ROWS_PER_TILE)],
                        out_hbm.at[c, pl.ds(row0, _ROWS_PER_TILE)])

    return sc_scatter


_sc_scatter = _sc_scatter_build()


def _edge_mm_body(h_ref, W_ref, b_ref, m2_ref):
    m = (
        jnp.dot(h_ref[...], W_ref[...], preferred_element_type=jnp.float32)
        + b_ref[...]
    )
    m2_ref[0] = m[:, :_DH]
    m2_ref[1] = m[:, _DH:]


def _gru(p_ref, h_ref, W_ih_ref, W_hh_ref, b_ih_ref, b_hh_ref):
    a_lo = p_ref[0]
    a_hi = p_ref[1]
    gi = (
        jnp.dot(a_lo, W_ih_ref[...][:_DH, :], preferred_element_type=jnp.float32)
        + jnp.dot(a_hi, W_ih_ref[...][_DH:, :], preferred_element_type=jnp.float32)
        + b_ih_ref[...]
    )
    h = h_ref[...]
    gh = jnp.dot(h, W_hh_ref[...], preferred_element_type=jnp.float32) + b_hh_ref[...]
    r = jax.nn.sigmoid(gi[:, :_D] + gh[:, :_D])
    z = jax.nn.sigmoid(gi[:, _D:2 * _D] + gh[:, _D:2 * _D])
    n = jnp.tanh(gi[:, 2 * _D:] + r * gh[:, 2 * _D:])
    return (1.0 - z) * n + z * h


def _gru_edge_body(p_ref, h_ref, W_ih_ref, W_hh_ref, b_ih_ref, b_hh_ref,
                   W_edge_ref, b_edge_ref, hn_ref, m2_ref):
    hn = _gru(p_ref, h_ref, W_ih_ref, W_hh_ref, b_ih_ref, b_hh_ref)
    hn_ref[...] = hn
    m = (
        jnp.dot(hn, W_edge_ref[...], preferred_element_type=jnp.float32)
        + b_edge_ref[...]
    )
    m2_ref[0] = m[:, :_DH]
    m2_ref[1] = m[:, _DH:]


def _gru_fc_body(p_ref, h_ref, W_ih_ref, W_hh_ref, b_ih_ref, b_hh_ref,
                 W_fc_ref, b_fc_ref, out_ref):
    hn = _gru(p_ref, h_ref, W_ih_ref, W_hh_ref, b_ih_ref, b_hh_ref)
    e = jnp.where(hn > 0, hn, jnp.exp(jnp.minimum(hn, 0.0)) - 1.0)
    out_ref[...] = (
        jnp.dot(e, W_fc_ref[...], preferred_element_type=jnp.float32)
        + b_fc_ref[...]
    )


def _full(shape):
    return pl.BlockSpec(shape, lambda i: tuple(0 for _ in shape))


_GRID = _N // _BN

_edge_mm = pl.pallas_call(
    _edge_mm_body,
    grid=(_GRID,),
    in_specs=[
        pl.BlockSpec((_BN, _D), lambda i: (i, 0)),
        _full((_D, _D)),
        _full((1, _D)),
    ],
    out_specs=pl.BlockSpec((2, _BN, _DH), lambda i: (0, i, 0)),
    out_shape=jax.ShapeDtypeStruct((2, _N, _DH), jnp.float32),
)

_gru_edge = pl.pallas_call(
    _gru_edge_body,
    grid=(_GRID,),
    in_specs=[
        pl.BlockSpec((2, _BN, _DH), lambda i: (0, i, 0)),
        pl.BlockSpec((_BN, _D), lambda i: (i, 0)),
        _full((_D, 3 * _D)),
        _full((_D, 3 * _D)),
        _full((1, 3 * _D)),
        _full((1, 3 * _D)),
        _full((_D, _D)),
        _full((1, _D)),
    ],
    out_specs=[
        pl.BlockSpec((_BN, _D), lambda i: (i, 0)),
        pl.BlockSpec((2, _BN, _DH), lambda i: (0, i, 0)),
    ],
    out_shape=[
        jax.ShapeDtypeStruct((_N, _D), jnp.float32),
        jax.ShapeDtypeStruct((2, _N, _DH), jnp.float32),
    ],
)

_gru_fc = pl.pallas_call(
    _gru_fc_body,
    grid=(_GRID,),
    in_specs=[
        pl.BlockSpec((2, _BN, _DH), lambda i: (0, i, 0)),
        pl.BlockSpec((_BN, _D), lambda i: (i, 0)),
        _full((_D, 3 * _D)),
        _full((_D, 3 * _D)),
        _full((1, 3 * _D)),
        _full((1, 3 * _D)),
        _full((_D, _NCLASS)),
        _full((1, _NCLASS)),
    ],
    out_specs=pl.BlockSpec((_BN, _NCLASS), lambda i: (i, 0)),
    out_shape=jax.ShapeDtypeStruct((_N, _NCLASS), jnp.float32),
)


def kernel(x, edge_index, W_edge, b_edge, W_ih, W_hh, b_ih, b_hh, W_fc, b_fc):
    src = edge_index[0].astype(jnp.int32).reshape(16, _NCH, _K)
    dst = edge_index[1].astype(jnp.int32).reshape(16, _NCH, _K)
    b_edge2 = b_edge.reshape(1, _D)
    b_ih2 = b_ih.reshape(1, 3 * _D)
    b_hh2 = b_hh.reshape(1, 3 * _D)
    b_fc2 = b_fc.reshape(1, _NCLASS)

    m1 = _edge_mm(x, W_edge, b_edge2)
    p1 = _sc_scatter(m1, src, dst)
    h1, m2 = _gru_edge(p1, x, W_ih, W_hh, b_ih2, b_hh2, W_edge, b_edge2)
    p2 = _sc_scatter(m2, src, dst)
    logits = _gru_fc(p2, h1, W_ih, W_hh, b_ih2, b_hh2, W_fc, b_fc2)
    return logits
